# baseline (speedup 1.0000x reference)
import jax
import jax.numpy as jnp
from jax.experimental import pallas as pl
from jax.experimental.pallas import tpu as pltpu

_LANES = 128
_BLK = 64
_BIG = 1 << 30


def _point_head_kern(bp_ref, pts_ref, out_ref):
    x = pts_ref[0]
    y = pts_ref[1]
    z = pts_ref[2]
    num_boxes = bp_ref.shape[0]

    keyacc = jnp.full(x.shape, _BIG, jnp.int32)
    extacc = jnp.zeros(x.shape, jnp.bool_)
    for b in range(num_boxes):
        cx = bp_ref[b, 0]
        cy = bp_ref[b, 1]
        cz = bp_ref[b, 2]
        ch = bp_ref[b, 3]
        sh = bp_ref[b, 4]
        hx = bp_ref[b, 5]
        hy = bp_ref[b, 6]
        hz = bp_ref[b, 7]
        hxe = bp_ref[b, 8]
        hye = bp_ref[b, 9]
        hze = bp_ref[b, 10]
        kb = bp_ref[b, 11].astype(jnp.int32)
        sx = x - cx
        sy = y - cy
        sz = z - cz
        lx = sx * ch + sy * sh
        ly = sy * ch - sx * sh
        ax = jnp.abs(lx)
        ay = jnp.abs(ly)
        az = jnp.abs(sz)
        in_gt = (ax < hx) & (ay < hy) & (az < hz)
        in_ext = (ax < hxe) & (ay < hye) & (az < hze)
        keyacc = jnp.minimum(keyacc, jnp.where(in_gt, kb, jnp.int32(_BIG)))
        extacc = extacc | in_ext
    fg = keyacc < _BIG
    out_ref[...] = jnp.where(fg, keyacc & 3,
                             jnp.where(extacc, jnp.int32(-1), jnp.int32(0)))


def kernel(points, gt_boxes, extend_gt_boxes):
    n = points.shape[0]
    rows = n // _LANES
    m = gt_boxes.shape[0]
    pts = points.T.reshape(3, rows, _LANES)
    keyf = (4.0 * jnp.arange(m, dtype=jnp.float32)
            + gt_boxes[:, 7])[:, None]
    bp = jnp.concatenate([
        gt_boxes[:, 0:3],
        jnp.cos(gt_boxes[:, 6])[:, None],
        jnp.sin(gt_boxes[:, 6])[:, None],
        gt_boxes[:, 3:6] / 2.0,
        extend_gt_boxes[:, 3:6] / 2.0,
        keyf,
    ], axis=1)
    out = pl.pallas_call(
        _point_head_kern,
        grid=(rows // _BLK,),
        in_specs=[
            pl.BlockSpec(memory_space=pltpu.SMEM),
            pl.BlockSpec((3, _BLK, _LANES), lambda i: (0, i, 0)),
        ],
        out_specs=pl.BlockSpec((_BLK, _LANES), lambda i: (i, 0)),
        out_shape=jax.ShapeDtypeStruct((rows, _LANES), jnp.int32),
        compiler_params=pltpu.CompilerParams(
            dimension_semantics=("parallel",)),
    )(bp, pts)
    return out.reshape(n)
